# baseline (speedup 1.0000x reference)
import functools

import jax
import jax.numpy as jnp
from jax import lax
from jax.experimental import pallas as pl
from jax.experimental.pallas import tpu as pltpu
from jax.experimental.pallas import tpu_sc as plsc

BATCH = 4096
SEQ = 200
D_MODEL = 64
SCALE = float(D_MODEL) ** 0.5
NC, NS, L = 2, 16, 16
NW = NC * NS
BW = BATCH // NW
NPAIR = SEQ // 2


def _body(xt_hbm, table_hbm, out_hbm,
          ibuf, rows0, rows1, ob0, ob1, gsem0, gsem1, ssem0, ssem1):
    wid = lax.axis_index("s") * NC + lax.axis_index("c")
    b0 = wid * BW
    iota = lax.iota(jnp.int32, L)

    pltpu.sync_copy(xt_hbm.at[:, pl.ds(b0, BW)], ibuf)

    def gather(t, rbuf, sem):
        return pltpu.async_copy(table_hbm.at[ibuf.at[t]], rbuf, sem)

    obase = [(g * L + iota) * BW for g in range(D_MODEL // L)]

    def transpose_scale(rbuf, obuf):
        @plsc.parallel_loop(0, BW, unroll=4)
        def _(b):
            for g in range(D_MODEL // L):
                v = rbuf[b, pl.ds(g * L, L)]
                plsc.store_scatter(obuf, [obase[g] + b], v * SCALE)

    def store(t, obuf, sem):
        for j in range(D_MODEL):
            pltpu.async_copy(
                obuf.at[pl.ds(j * BW, BW)],
                out_hbm.at[t, j, pl.ds(b0, BW)], sem)

    def store_wait(obuf, sem):
        for j in range(D_MODEL):
            pltpu.make_async_copy(
                obuf.at[pl.ds(j * BW, BW)],
                out_hbm.at[0, j, pl.ds(b0, BW)], sem).wait()

    def pair(p, carry):
        t0 = 2 * p
        t1 = 2 * p + 1

        @pl.when(p > 0)
        def _():
            store_wait(ob0, ssem0)

        g0 = gather(t0, rows0, gsem0)

        @pl.when(p > 0)
        def _():
            store_wait(ob1, ssem1)

        g1 = gather(t1, rows1, gsem1)

        g0.wait()
        transpose_scale(rows0, ob0)
        store(t0, ob0, ssem0)

        g1.wait()
        transpose_scale(rows1, ob1)
        store(t1, ob1, ssem1)
        return carry

    lax.fori_loop(0, NPAIR, pair, 0)
    store_wait(ob0, ssem0)
    store_wait(ob1, ssem1)


@jax.jit
def _run(x, table):
    mesh = plsc.VectorSubcoreMesh(core_axis_name="c", subcore_axis_name="s")
    f = functools.partial(
        pl.kernel,
        out_type=jax.ShapeDtypeStruct((SEQ, D_MODEL, BATCH), jnp.float32),
        mesh=mesh,
        compiler_params=pltpu.CompilerParams(
            use_tc_tiling_on_sc=False, needs_layout_passes=False),
        scratch_types=[
            pltpu.VMEM((SEQ, BW), jnp.int32),
            pltpu.VMEM((BW, D_MODEL), jnp.float32),
            pltpu.VMEM((BW, D_MODEL), jnp.float32),
            pltpu.VMEM((D_MODEL * BW,), jnp.float32),
            pltpu.VMEM((D_MODEL * BW,), jnp.float32),
            pltpu.SemaphoreType.DMA,
            pltpu.SemaphoreType.DMA,
            pltpu.SemaphoreType.DMA,
            pltpu.SemaphoreType.DMA,
        ],
    )(_body)
    out_phys = f(x.T, table)
    return out_phys.transpose(2, 0, 1)


def kernel(x, table):
    return _run(x, table)
